# Initial kernel scaffold; baseline (speedup 1.0000x reference)
import jax
import jax.numpy as jnp
from jax.experimental import pallas as pl


def kernel(x, edge_index, W1, b1, W2, b2, Wp1, bp1, Wp2, bp2):
    raise NotImplementedError("write your pallas kernel here")


# speedup vs baseline: 11.2283x; 11.2283x over previous
import functools

import jax
import jax.numpy as jnp
from jax import lax
from jax.experimental import pallas as pl
from jax.experimental.pallas import tpu as pltpu
from jax.experimental.pallas import tpu_sc as plsc

NC = 2
NS = 16
NW = NC * NS

CK = 80
HH = 64


def _mesh():
    return plsc.VectorSubcoreMesh(core_axis_name="c", subcore_axis_name="s")


@functools.cache
def _deg_fn(N, E):
    EPW = E // NW

    @functools.partial(
        pl.kernel, mesh=_mesh(),
        compiler_params=pltpu.CompilerParams(needs_layout_passes=False),
        out_type=jax.ShapeDtypeStruct((NW, N), jnp.float32),
        scratch_types=[
            pltpu.VMEM((EPW,), jnp.int32),
            pltpu.VMEM((N,), jnp.float32),
        ],
    )
    def deg_k(row_hbm, out_hbm, idx_v, deg_v):
        c = lax.axis_index("c")
        s = lax.axis_index("s")
        wid = s * NC + c
        pltpu.sync_copy(row_hbm.at[wid], idx_v)
        zero16 = jnp.zeros((16,), jnp.float32)

        def zbody(i, carry):
            deg_v[pl.ds(i * 16, 16)] = zero16
            return carry

        lax.fori_loop(0, N // 16, zbody, 0)
        ones16 = jnp.ones((16,), jnp.float32)

        def hbody(i, carry):
            idx16 = idx_v[pl.ds(i * 16, 16)]
            plsc.addupdate_scatter(deg_v, [idx16], ones16)
            return carry

        lax.fori_loop(0, EPW // 16, hbody, 0)
        pltpu.sync_copy(deg_v, out_hbm.at[wid])

    return deg_k


@functools.cache
def _spmm_fn(N, E):
    EPW = E // NW
    NCH = EPW // CK
    RPT = N // NS
    ZR = 125
    NZ = RPT // ZR

    @functools.partial(
        pl.kernel, mesh=_mesh(),
        compiler_params=pltpu.CompilerParams(
            needs_layout_passes=False, use_tc_tiling_on_sc=False),
        out_type=jax.ShapeDtypeStruct((NC, N, HH), jnp.float32),
        scratch_types=[
            pltpu.VMEM((NCH, CK), jnp.int32),
            pltpu.VMEM((NCH, CK), jnp.int32),
            pltpu.VMEM((CK, HH), jnp.float32),
            pltpu.VMEM((ZR, HH), jnp.float32),
            pltpu.VMEM_SHARED((N, HH), jnp.float32),
            pltpu.SemaphoreType.DMA,
        ],
    )
    def spmm_k(g_hbm, row_hbm, col_hbm, out_hbm,
               row_v, col_v, rows_v, zbuf, acc, sem):
        c = lax.axis_index("c")
        s = lax.axis_index("s")
        wid = s * NC + c
        pltpu.sync_copy(row_hbm.at[wid], row_v)
        pltpu.sync_copy(col_hbm.at[wid], col_v)

        zero16 = jnp.zeros((16,), jnp.float32)

        def zrow(i, carry):
            def zcol(k, carry2):
                zbuf[i, pl.ds(k * 16, 16)] = zero16
                return carry2
            return lax.fori_loop(0, HH // 16, zcol, carry)

        lax.fori_loop(0, ZR, zrow, 0)

        def zacc(i, carry):
            pltpu.sync_copy(zbuf, acc.at[pl.ds(s * RPT + i * ZR, ZR)])
            return carry

        lax.fori_loop(0, NZ, zacc, 0)
        plsc.subcore_barrier()

        def ebody(j, carry):
            pltpu.async_copy(g_hbm.at[col_v.at[j]], rows_v, sem).wait()
            pltpu.sync_copy(rows_v, acc.at[row_v.at[j]], add=True)
            return carry

        lax.fori_loop(0, NCH, ebody, 0)
        plsc.subcore_barrier()
        DR = (N // NS) // 8 * 8
        pltpu.sync_copy(acc.at[pl.ds(s * DR, DR)],
                        out_hbm.at[c, pl.ds(s * DR, DR)])
        TAIL = N - NS * DR

        @pl.when(s == 0)
        def _():
            pltpu.sync_copy(acc.at[pl.ds(NS * DR, TAIL)],
                            out_hbm.at[c, pl.ds(NS * DR, TAIL)])

    return spmm_k


def _prep(parts, x):
    N, D = x.shape

    def body(parts_ref, x_ref, dinv_ref, gl_ref, gr_ref):
        deg = jnp.sum(parts_ref[...], axis=0)
        dinv = lax.rsqrt(deg)
        dinv_ref[...] = dinv
        g = x_ref[...] * dinv[:, None]
        gl_ref[...] = g[:, :HH]
        gr_ref[...] = g[:, HH:]

    return pl.pallas_call(
        body,
        out_shape=[
            jax.ShapeDtypeStruct((N,), jnp.float32),
            jax.ShapeDtypeStruct((N, HH), jnp.float32),
            jax.ShapeDtypeStruct((N, HH), jnp.float32),
        ],
    )(parts, x)


def _layer1(pl_parts, pr_parts, dinv, W1, b1):
    N = pl_parts.shape[1]
    H = W1.shape[1]
    BN = 1000

    def body(pl_ref, pr_ref, dinv_ref, W_ref, b_ref, gl_ref, gr_ref):
        dv = dinv_ref[...][:, None]
        sl = (pl_ref[0] + pl_ref[1]) * dv
        sr = (pr_ref[0] + pr_ref[1]) * dv
        srows = jnp.concatenate([sl, sr], axis=1)
        h = jnp.dot(srows, W_ref[...], preferred_element_type=jnp.float32)
        h = jnp.maximum(h + b_ref[...][None, :], 0.0)
        g = h * dv
        gl_ref[...] = g[:, :HH]
        gr_ref[...] = g[:, HH:]

    return pl.pallas_call(
        body,
        out_shape=[
            jax.ShapeDtypeStruct((N, HH), jnp.float32),
            jax.ShapeDtypeStruct((N, HH), jnp.float32),
        ],
    )(pl_parts, pr_parts, dinv, W1, b1)


def _final(pl_parts, pr_parts, dinv, W2, b2, Wp1, bp1, Wp2, bp2):
    N = pl_parts.shape[1]
    O = Wp2.shape[1]
    BN = 1000

    def body(pl_ref, pr_ref, dinv_ref, W2_ref, b2_ref,
             Wp1_ref, bp1_ref, Wp2_ref, bp2_ref, out_ref):
        dv = dinv_ref[...][:, None]
        sl = (pl_ref[0] + pl_ref[1]) * dv
        sr = (pr_ref[0] + pr_ref[1]) * dv
        srows = jnp.concatenate([sl, sr], axis=1)
        h = jnp.dot(srows, W2_ref[...], preferred_element_type=jnp.float32)
        h = jnp.maximum(h + b2_ref[...][None, :], 0.0)
        h = jnp.dot(h, Wp1_ref[...], preferred_element_type=jnp.float32)
        h = h + bp1_ref[...][None, :]
        h = jnp.dot(h, Wp2_ref[...], preferred_element_type=jnp.float32)
        h = h + bp2_ref[...][None, :]
        m = jnp.max(h, axis=1, keepdims=True)
        lse = jnp.log(jnp.sum(jnp.exp(h - m), axis=1, keepdims=True)) + m
        out_ref[...] = h - lse

    return pl.pallas_call(
        body,
        out_shape=jax.ShapeDtypeStruct((N, O), jnp.float32),
    )(pl_parts, pr_parts, dinv, W2, b2, Wp1, bp1, Wp2, bp2)


def kernel(x, edge_index, W1, b1, W2, b2, Wp1, bp1, Wp2, bp2):
    N, D = x.shape
    E = edge_index.shape[1]
    EPW = E // NW
    NCH = EPW // CK

    row = edge_index[0]
    col = edge_index[1]
    row_r = row.reshape(NW, NCH, CK)
    col_r = col.reshape(NW, NCH, CK)
    row_flat = row.reshape(NW, EPW)

    spmm = _spmm_fn(N, E)

    deg_parts = _deg_fn(N, E)(row_flat)
    dinv, g1l, g1r = _prep(deg_parts, x)

    p1l = spmm(g1l, row_r, col_r)
    p1r = spmm(g1r, row_r, col_r)
    g2l, g2r = _layer1(p1l, p1r, dinv, W1, b1)

    p2l = spmm(g2l, row_r, col_r)
    p2r = spmm(g2r, row_r, col_r)
    return _final(p2l, p2r, dinv, W2, b2, Wp1, bp1, Wp2, bp2)


# speedup vs baseline: 17.7491x; 1.5807x over previous
import functools

import jax
import jax.numpy as jnp
from jax import lax
from jax.experimental import pallas as pl
from jax.experimental.pallas import tpu as pltpu
from jax.experimental.pallas import tpu_sc as plsc

NC = 2
NS = 16
NW = NC * NS

CK = 80
HH = 64


def _mesh():
    return plsc.VectorSubcoreMesh(core_axis_name="c", subcore_axis_name="s")


@functools.cache
def _deg_fn(N, E):
    EPW = E // NW

    @functools.partial(
        pl.kernel, mesh=_mesh(),
        compiler_params=pltpu.CompilerParams(needs_layout_passes=False),
        out_type=jax.ShapeDtypeStruct((NW, N), jnp.float32),
        scratch_types=[
            pltpu.VMEM((EPW,), jnp.int32),
            pltpu.VMEM((N,), jnp.float32),
        ],
    )
    def deg_k(row_hbm, out_hbm, idx_v, deg_v):
        c = lax.axis_index("c")
        s = lax.axis_index("s")
        wid = s * NC + c
        pltpu.sync_copy(row_hbm.at[wid], idx_v)
        zero16 = jnp.zeros((16,), jnp.float32)

        def zbody(i, carry):
            deg_v[pl.ds(i * 16, 16)] = zero16
            return carry

        lax.fori_loop(0, N // 16, zbody, 0)
        ones16 = jnp.ones((16,), jnp.float32)

        def hbody(i, carry):
            idx16 = idx_v[pl.ds(i * 16, 16)]
            plsc.addupdate_scatter(deg_v, [idx16], ones16)
            return carry

        lax.fori_loop(0, EPW // 16, hbody, 0)
        pltpu.sync_copy(deg_v, out_hbm.at[wid])

    return deg_k


@functools.cache
def _spmm_fn(N, E):
    EPW = E // NW
    NCH = EPW // CK
    RPT = N // NS
    ZR = 125
    NZ = RPT // ZR

    @functools.partial(
        pl.kernel, mesh=_mesh(),
        compiler_params=pltpu.CompilerParams(
            needs_layout_passes=False, use_tc_tiling_on_sc=False),
        out_type=jax.ShapeDtypeStruct((NC, N, HH), jnp.float32),
        scratch_types=[
            pltpu.VMEM((NCH, CK), jnp.int32),
            pltpu.VMEM((NCH, CK), jnp.int32),
            pltpu.VMEM((CK, HH), jnp.float32),
            pltpu.VMEM((CK, HH), jnp.float32),
            pltpu.VMEM((ZR, HH), jnp.float32),
            pltpu.VMEM_SHARED((N, HH), jnp.float32),
            pltpu.SemaphoreType.DMA,
            pltpu.SemaphoreType.DMA,
        ],
    )
    def spmm_k(g_hbm, row_hbm, col_hbm, out_hbm,
               row_v, col_v, rows0_v, rows1_v, zbuf, acc, sem0, sem1):
        c = lax.axis_index("c")
        s = lax.axis_index("s")
        wid = s * NC + c
        pltpu.sync_copy(row_hbm.at[wid], row_v)
        pltpu.sync_copy(col_hbm.at[wid], col_v)

        zero16 = jnp.zeros((16,), jnp.float32)

        def zrow(i, carry):
            def zcol(k, carry2):
                zbuf[i, pl.ds(k * 16, 16)] = zero16
                return carry2
            return lax.fori_loop(0, HH // 16, zcol, carry)

        lax.fori_loop(0, ZR, zrow, 0)

        def zacc(i, carry):
            pltpu.sync_copy(zbuf, acc.at[pl.ds(s * RPT + i * ZR, ZR)])
            return carry

        lax.fori_loop(0, NZ, zacc, 0)
        plsc.subcore_barrier()

        def start(j, buf, sem_):
            pltpu.async_copy(g_hbm.at[col_v.at[j]], buf, sem_)

        def finish(j, buf, sem_):
            pltpu.make_async_copy(g_hbm.at[col_v.at[j]], buf, sem_).wait()
            pltpu.sync_copy(buf, acc.at[row_v.at[j]], add=True)

        start(0, rows0_v, sem0)
        start(1, rows1_v, sem1)

        def ebody(i, carry):
            j0 = 2 * i
            finish(j0, rows0_v, sem0)

            @pl.when(j0 + 2 < NCH)
            def _():
                start(j0 + 2, rows0_v, sem0)

            finish(j0 + 1, rows1_v, sem1)

            @pl.when(j0 + 3 < NCH)
            def _():
                start(j0 + 3, rows1_v, sem1)

            return carry

        lax.fori_loop(0, NCH // 2, ebody, 0)
        if NCH % 2:
            finish(NCH - 1, rows0_v, sem0)
        plsc.subcore_barrier()
        DR = (N // NS) // 8 * 8
        pltpu.sync_copy(acc.at[pl.ds(s * DR, DR)],
                        out_hbm.at[c, pl.ds(s * DR, DR)])
        TAIL = N - NS * DR

        @pl.when(s == 0)
        def _():
            pltpu.sync_copy(acc.at[pl.ds(NS * DR, TAIL)],
                            out_hbm.at[c, pl.ds(NS * DR, TAIL)])

    return spmm_k


def _prep(parts, x):
    N, D = x.shape

    def body(parts_ref, x_ref, dinv_ref, gl_ref, gr_ref):
        deg = jnp.sum(parts_ref[...], axis=0)
        dinv = lax.rsqrt(deg)
        dinv_ref[...] = dinv
        g = x_ref[...] * dinv[:, None]
        gl_ref[...] = g[:, :HH]
        gr_ref[...] = g[:, HH:]

    return pl.pallas_call(
        body,
        out_shape=[
            jax.ShapeDtypeStruct((N,), jnp.float32),
            jax.ShapeDtypeStruct((N, HH), jnp.float32),
            jax.ShapeDtypeStruct((N, HH), jnp.float32),
        ],
    )(parts, x)


def _layer1(pl_parts, pr_parts, dinv, W1, b1):
    N = pl_parts.shape[1]
    H = W1.shape[1]
    BN = 1000

    def body(pl_ref, pr_ref, dinv_ref, W_ref, b_ref, gl_ref, gr_ref):
        dv = dinv_ref[...][:, None]
        sl = (pl_ref[0] + pl_ref[1]) * dv
        sr = (pr_ref[0] + pr_ref[1]) * dv
        srows = jnp.concatenate([sl, sr], axis=1)
        h = jnp.dot(srows, W_ref[...], preferred_element_type=jnp.float32)
        h = jnp.maximum(h + b_ref[...][None, :], 0.0)
        g = h * dv
        gl_ref[...] = g[:, :HH]
        gr_ref[...] = g[:, HH:]

    return pl.pallas_call(
        body,
        out_shape=[
            jax.ShapeDtypeStruct((N, HH), jnp.float32),
            jax.ShapeDtypeStruct((N, HH), jnp.float32),
        ],
    )(pl_parts, pr_parts, dinv, W1, b1)


def _final(pl_parts, pr_parts, dinv, W2, b2, Wp1, bp1, Wp2, bp2):
    N = pl_parts.shape[1]
    O = Wp2.shape[1]
    BN = 1000

    def body(pl_ref, pr_ref, dinv_ref, W2_ref, b2_ref,
             Wp1_ref, bp1_ref, Wp2_ref, bp2_ref, out_ref):
        dv = dinv_ref[...][:, None]
        sl = (pl_ref[0] + pl_ref[1]) * dv
        sr = (pr_ref[0] + pr_ref[1]) * dv
        srows = jnp.concatenate([sl, sr], axis=1)
        h = jnp.dot(srows, W2_ref[...], preferred_element_type=jnp.float32)
        h = jnp.maximum(h + b2_ref[...][None, :], 0.0)
        h = jnp.dot(h, Wp1_ref[...], preferred_element_type=jnp.float32)
        h = h + bp1_ref[...][None, :]
        h = jnp.dot(h, Wp2_ref[...], preferred_element_type=jnp.float32)
        h = h + bp2_ref[...][None, :]
        m = jnp.max(h, axis=1, keepdims=True)
        lse = jnp.log(jnp.sum(jnp.exp(h - m), axis=1, keepdims=True)) + m
        out_ref[...] = h - lse

    return pl.pallas_call(
        body,
        out_shape=jax.ShapeDtypeStruct((N, O), jnp.float32),
    )(pl_parts, pr_parts, dinv, W2, b2, Wp1, bp1, Wp2, bp2)


def kernel(x, edge_index, W1, b1, W2, b2, Wp1, bp1, Wp2, bp2):
    N, D = x.shape
    E = edge_index.shape[1]
    EPW = E // NW
    NCH = EPW // CK

    row = edge_index[0]
    col = edge_index[1]
    row_r = row.reshape(NW, NCH, CK)
    col_r = col.reshape(NW, NCH, CK)
    row_flat = row.reshape(NW, EPW)

    spmm = _spmm_fn(N, E)

    deg_parts = _deg_fn(N, E)(row_flat)
    dinv, g1l, g1r = _prep(deg_parts, x)

    p1l = spmm(g1l, row_r, col_r)
    p1r = spmm(g1r, row_r, col_r)
    g2l, g2r = _layer1(p1l, p1r, dinv, W1, b1)

    p2l = spmm(g2l, row_r, col_r)
    p2r = spmm(g2r, row_r, col_r)
    return _final(p2l, p2r, dinv, W2, b2, Wp1, bp1, Wp2, bp2)


# speedup vs baseline: 23.8239x; 1.3423x over previous
import functools

import jax
import jax.numpy as jnp
from jax import lax
from jax.experimental import pallas as pl
from jax.experimental.pallas import tpu as pltpu
from jax.experimental.pallas import tpu_sc as plsc

NC = 2
NS = 16
NW = NC * NS

CK = 80
HH = 64


def _mesh():
    return plsc.VectorSubcoreMesh(core_axis_name="c", subcore_axis_name="s")


@functools.cache
def _deg_fn(N, E):
    EPW = E // NW

    @functools.partial(
        pl.kernel, mesh=_mesh(),
        compiler_params=pltpu.CompilerParams(needs_layout_passes=False),
        out_type=jax.ShapeDtypeStruct((NW, N), jnp.float32),
        scratch_types=[
            pltpu.VMEM((EPW,), jnp.int32),
            pltpu.VMEM((N,), jnp.float32),
        ],
    )
    def deg_k(row_hbm, out_hbm, idx_v, deg_v):
        c = lax.axis_index("c")
        s = lax.axis_index("s")
        wid = s * NC + c
        pltpu.sync_copy(row_hbm.at[wid], idx_v)
        zero16 = jnp.zeros((16,), jnp.float32)

        def zbody(i, carry):
            deg_v[pl.ds(i * 16, 16)] = zero16
            return carry

        lax.fori_loop(0, N // 16, zbody, 0)
        ones16 = jnp.ones((16,), jnp.float32)

        def hbody(i, carry):
            idx16 = idx_v[pl.ds(i * 16, 16)]
            plsc.addupdate_scatter(deg_v, [idx16], ones16)
            return carry

        lax.fori_loop(0, EPW // 16, hbody, 0)
        pltpu.sync_copy(deg_v, out_hbm.at[wid])

    return deg_k


@functools.cache
def _spmm_fn(N, E):
    EPW = E // NW
    NCH = EPW // CK
    RPT = N // NS
    ZR = 125
    NZ = RPT // ZR

    @functools.partial(
        pl.kernel, mesh=_mesh(),
        compiler_params=pltpu.CompilerParams(
            needs_layout_passes=False, use_tc_tiling_on_sc=False),
        out_type=jax.ShapeDtypeStruct((NC, N, HH), jnp.float32),
        scratch_types=[
            pltpu.VMEM((NCH, CK), jnp.int32),
            pltpu.VMEM((NCH, CK), jnp.int32),
            [pltpu.VMEM((CK, HH), jnp.float32)] * 4,
            pltpu.VMEM((ZR, HH), jnp.float32),
            pltpu.VMEM_SHARED((N, HH), jnp.float32),
            [pltpu.SemaphoreType.DMA] * 4,
        ],
    )
    def spmm_k(g_hbm, row_hbm, col_hbm, out_hbm,
               row_v, col_v, rbufs, zbuf, acc, sems):
        c = lax.axis_index("c")
        s = lax.axis_index("s")
        wid = s * NC + c
        pltpu.sync_copy(row_hbm.at[wid], row_v)
        pltpu.sync_copy(col_hbm.at[wid], col_v)

        zero16 = jnp.zeros((16,), jnp.float32)

        def zrow(i, carry):
            def zcol(k, carry2):
                zbuf[i, pl.ds(k * 16, 16)] = zero16
                return carry2
            return lax.fori_loop(0, HH // 16, zcol, carry)

        lax.fori_loop(0, ZR, zrow, 0)

        def zacc(i, carry):
            pltpu.sync_copy(zbuf, acc.at[pl.ds(s * RPT + i * ZR, ZR)])
            return carry

        lax.fori_loop(0, NZ, zacc, 0)
        plsc.subcore_barrier()

        NB = 4

        def start(j, b):
            pltpu.async_copy(g_hbm.at[col_v.at[j]], rbufs[b], sems[b])

        def finish(j, b):
            pltpu.make_async_copy(g_hbm.at[col_v.at[j]], rbufs[b],
                                  sems[b]).wait()
            pltpu.sync_copy(rbufs[b], acc.at[row_v.at[j]], add=True)

        for b in range(NB):
            start(b, b)

        def ebody(i, carry):
            j0 = NB * i
            for b in range(NB):
                finish(j0 + b, b)

                @pl.when(j0 + b + NB < NCH)
                def _():
                    start(j0 + b + NB, b)

            return carry

        lax.fori_loop(0, NCH // NB, ebody, 0)
        for t in range(NCH - NCH % NB, NCH):
            finish(t, t % NB)
        plsc.subcore_barrier()
        DR = (N // NS) // 8 * 8
        pltpu.sync_copy(acc.at[pl.ds(s * DR, DR)],
                        out_hbm.at[c, pl.ds(s * DR, DR)])
        TAIL = N - NS * DR

        @pl.when(s == 0)
        def _():
            pltpu.sync_copy(acc.at[pl.ds(NS * DR, TAIL)],
                            out_hbm.at[c, pl.ds(NS * DR, TAIL)])

    return spmm_k


def _prep(parts, x):
    N, D = x.shape

    def body(parts_ref, x_ref, dinv_ref, gl_ref, gr_ref):
        deg = jnp.sum(parts_ref[...], axis=0)
        dinv = lax.rsqrt(deg)
        dinv_ref[...] = dinv
        g = x_ref[...] * dinv[:, None]
        gl_ref[...] = g[:, :HH]
        gr_ref[...] = g[:, HH:]

    return pl.pallas_call(
        body,
        out_shape=[
            jax.ShapeDtypeStruct((N,), jnp.float32),
            jax.ShapeDtypeStruct((N, HH), jnp.float32),
            jax.ShapeDtypeStruct((N, HH), jnp.float32),
        ],
    )(parts, x)


def _layer1(pl_parts, pr_parts, dinv, W1, b1):
    N = pl_parts.shape[1]
    H = W1.shape[1]
    BN = 1000

    def body(pl_ref, pr_ref, dinv_ref, W_ref, b_ref, gl_ref, gr_ref):
        dv = dinv_ref[...][:, None]
        sl = (pl_ref[0] + pl_ref[1]) * dv
        sr = (pr_ref[0] + pr_ref[1]) * dv
        srows = jnp.concatenate([sl, sr], axis=1)
        h = jnp.dot(srows, W_ref[...], preferred_element_type=jnp.float32)
        h = jnp.maximum(h + b_ref[...][None, :], 0.0)
        g = h * dv
        gl_ref[...] = g[:, :HH]
        gr_ref[...] = g[:, HH:]

    return pl.pallas_call(
        body,
        out_shape=[
            jax.ShapeDtypeStruct((N, HH), jnp.float32),
            jax.ShapeDtypeStruct((N, HH), jnp.float32),
        ],
    )(pl_parts, pr_parts, dinv, W1, b1)


def _final(pl_parts, pr_parts, dinv, W2, b2, Wp1, bp1, Wp2, bp2):
    N = pl_parts.shape[1]
    O = Wp2.shape[1]
    BN = 1000

    def body(pl_ref, pr_ref, dinv_ref, W2_ref, b2_ref,
             Wp1_ref, bp1_ref, Wp2_ref, bp2_ref, out_ref):
        dv = dinv_ref[...][:, None]
        sl = (pl_ref[0] + pl_ref[1]) * dv
        sr = (pr_ref[0] + pr_ref[1]) * dv
        srows = jnp.concatenate([sl, sr], axis=1)
        h = jnp.dot(srows, W2_ref[...], preferred_element_type=jnp.float32)
        h = jnp.maximum(h + b2_ref[...][None, :], 0.0)
        h = jnp.dot(h, Wp1_ref[...], preferred_element_type=jnp.float32)
        h = h + bp1_ref[...][None, :]
        h = jnp.dot(h, Wp2_ref[...], preferred_element_type=jnp.float32)
        h = h + bp2_ref[...][None, :]
        m = jnp.max(h, axis=1, keepdims=True)
        lse = jnp.log(jnp.sum(jnp.exp(h - m), axis=1, keepdims=True)) + m
        out_ref[...] = h - lse

    return pl.pallas_call(
        body,
        out_shape=jax.ShapeDtypeStruct((N, O), jnp.float32),
    )(pl_parts, pr_parts, dinv, W2, b2, Wp1, bp1, Wp2, bp2)


def kernel(x, edge_index, W1, b1, W2, b2, Wp1, bp1, Wp2, bp2):
    N, D = x.shape
    E = edge_index.shape[1]
    EPW = E // NW
    NCH = EPW // CK

    row = edge_index[0]
    col = edge_index[1]
    row_r = row.reshape(NW, NCH, CK)
    col_r = col.reshape(NW, NCH, CK)
    row_flat = row.reshape(NW, EPW)

    spmm = _spmm_fn(N, E)

    deg_parts = _deg_fn(N, E)(row_flat)
    dinv, g1l, g1r = _prep(deg_parts, x)

    p1l = spmm(g1l, row_r, col_r)
    p1r = spmm(g1r, row_r, col_r)
    g2l, g2r = _layer1(p1l, p1r, dinv, W1, b1)

    p2l = spmm(g2l, row_r, col_r)
    p2r = spmm(g2r, row_r, col_r)
    return _final(p2l, p2r, dinv, W2, b2, Wp1, bp1, Wp2, bp2)


# speedup vs baseline: 26.1794x; 1.0989x over previous
import functools

import jax
import jax.numpy as jnp
from jax import lax
from jax.experimental import pallas as pl
from jax.experimental.pallas import tpu as pltpu
from jax.experimental.pallas import tpu_sc as plsc

NC = 2
NS = 16
NW = NC * NS

CK = 80
HH = 64


def _mesh():
    return plsc.VectorSubcoreMesh(core_axis_name="c", subcore_axis_name="s")


@functools.cache
def _deg_fn(N, E):
    EPW = E // NW

    @functools.partial(
        pl.kernel, mesh=_mesh(),
        compiler_params=pltpu.CompilerParams(needs_layout_passes=False),
        out_type=jax.ShapeDtypeStruct((NW, N), jnp.float32),
        scratch_types=[
            pltpu.VMEM((EPW,), jnp.int32),
            pltpu.VMEM((N,), jnp.float32),
        ],
    )
    def deg_k(row_hbm, out_hbm, idx_v, deg_v):
        c = lax.axis_index("c")
        s = lax.axis_index("s")
        wid = s * NC + c
        pltpu.sync_copy(row_hbm.at[wid], idx_v)
        zero16 = jnp.zeros((16,), jnp.float32)

        def zbody(i, carry):
            deg_v[pl.ds(i * 16, 16)] = zero16
            return carry

        lax.fori_loop(0, N // 16, zbody, 0)
        ones16 = jnp.ones((16,), jnp.float32)

        def hbody(i, carry):
            idx16 = idx_v[pl.ds(i * 16, 16)]
            plsc.addupdate_scatter(deg_v, [idx16], ones16)
            return carry

        lax.fori_loop(0, EPW // 16, hbody, 0)
        pltpu.sync_copy(deg_v, out_hbm.at[wid])

    return deg_k


@functools.cache
def _spmm_fn(N, E):
    EPW = E // NS
    NCH = EPW // CK
    RPT = N // NS
    ZR = 125
    NZ = RPT // ZR

    @functools.partial(
        pl.kernel, mesh=_mesh(),
        compiler_params=pltpu.CompilerParams(
            needs_layout_passes=False, use_tc_tiling_on_sc=False),
        out_type=jax.ShapeDtypeStruct((NC, N, HH), jnp.float32),
        scratch_types=[
            pltpu.VMEM((NCH, CK), jnp.int32),
            pltpu.VMEM((NCH, CK), jnp.int32),
            [pltpu.VMEM((CK, HH), jnp.float32)] * 4,
            pltpu.VMEM((ZR, HH), jnp.float32),
            pltpu.VMEM_SHARED((N, HH), jnp.float32),
            [pltpu.SemaphoreType.DMA] * 4,
        ],
    )
    def spmm_k(g_hbm, row_hbm, col_hbm, out_hbm,
               row_v, col_v, rbufs, zbuf, acc, sems):
        c = lax.axis_index("c")
        s = lax.axis_index("s")
        pltpu.sync_copy(row_hbm.at[s], row_v)
        pltpu.sync_copy(col_hbm.at[s], col_v)
        ghalf = g_hbm.at[c]

        zero16 = jnp.zeros((16,), jnp.float32)

        def zrow(i, carry):
            def zcol(k, carry2):
                zbuf[i, pl.ds(k * 16, 16)] = zero16
                return carry2
            return lax.fori_loop(0, HH // 16, zcol, carry)

        lax.fori_loop(0, ZR, zrow, 0)

        def zacc(i, carry):
            pltpu.sync_copy(zbuf, acc.at[pl.ds(s * RPT + i * ZR, ZR)])
            return carry

        lax.fori_loop(0, NZ, zacc, 0)
        plsc.subcore_barrier()

        NB = 4

        def start(j, b):
            pltpu.async_copy(ghalf.at[col_v.at[j]], rbufs[b], sems[b])

        def finish(j, b):
            pltpu.make_async_copy(ghalf.at[col_v.at[j]], rbufs[b],
                                  sems[b]).wait()
            pltpu.sync_copy(rbufs[b], acc.at[row_v.at[j]], add=True)

        for b in range(NB):
            start(b, b)

        def ebody(i, carry):
            j0 = NB * i
            for b in range(NB):
                finish(j0 + b, b)

                @pl.when(j0 + b + NB < NCH)
                def _():
                    start(j0 + b + NB, b)

            return carry

        lax.fori_loop(0, NCH // NB, ebody, 0)
        for t in range(NCH - NCH % NB, NCH):
            finish(t, t % NB)
        plsc.subcore_barrier()
        DR = (N // NS) // 8 * 8
        pltpu.sync_copy(acc.at[pl.ds(s * DR, DR)],
                        out_hbm.at[c, pl.ds(s * DR, DR)])
        TAIL = N - NS * DR

        @pl.when(s == 0)
        def _():
            pltpu.sync_copy(acc.at[pl.ds(NS * DR, TAIL)],
                            out_hbm.at[c, pl.ds(NS * DR, TAIL)])

    return spmm_k


def _prep(parts, x):
    N, D = x.shape

    def body(parts_ref, x_ref, dinv_ref, g_ref):
        deg = jnp.sum(parts_ref[...], axis=0)
        dinv = lax.rsqrt(deg)
        dinv_ref[...] = dinv
        g = x_ref[...] * dinv[:, None]
        g_ref[0] = g[:, :HH]
        g_ref[1] = g[:, HH:]

    return pl.pallas_call(
        body,
        out_shape=[
            jax.ShapeDtypeStruct((N,), jnp.float32),
            jax.ShapeDtypeStruct((NC, N, HH), jnp.float32),
        ],
    )(parts, x)


def _layer1(parts, dinv, W1, b1):
    N = parts.shape[1]

    def body(p_ref, dinv_ref, W_ref, b_ref, g_ref):
        dv = dinv_ref[...][:, None]
        srows = jnp.concatenate([p_ref[0] * dv, p_ref[1] * dv], axis=1)
        h = jnp.dot(srows, W_ref[...], preferred_element_type=jnp.float32)
        h = jnp.maximum(h + b_ref[...][None, :], 0.0)
        g = h * dv
        g_ref[0] = g[:, :HH]
        g_ref[1] = g[:, HH:]

    return pl.pallas_call(
        body,
        out_shape=jax.ShapeDtypeStruct((NC, N, HH), jnp.float32),
    )(parts, dinv, W1, b1)


def _final(parts, dinv, W2, b2, Wp1, bp1, Wp2, bp2):
    N = parts.shape[1]
    O = Wp2.shape[1]

    def body(p_ref, dinv_ref, W2_ref, b2_ref,
             Wp1_ref, bp1_ref, Wp2_ref, bp2_ref, out_ref):
        dv = dinv_ref[...][:, None]
        srows = jnp.concatenate([p_ref[0] * dv, p_ref[1] * dv], axis=1)
        h = jnp.dot(srows, W2_ref[...], preferred_element_type=jnp.float32)
        h = jnp.maximum(h + b2_ref[...][None, :], 0.0)
        h = jnp.dot(h, Wp1_ref[...], preferred_element_type=jnp.float32)
        h = h + bp1_ref[...][None, :]
        h = jnp.dot(h, Wp2_ref[...], preferred_element_type=jnp.float32)
        h = h + bp2_ref[...][None, :]
        m = jnp.max(h, axis=1, keepdims=True)
        lse = jnp.log(jnp.sum(jnp.exp(h - m), axis=1, keepdims=True)) + m
        out_ref[...] = h - lse

    return pl.pallas_call(
        body,
        out_shape=jax.ShapeDtypeStruct((N, O), jnp.float32),
    )(parts, dinv, W2, b2, Wp1, bp1, Wp2, bp2)


def kernel(x, edge_index, W1, b1, W2, b2, Wp1, bp1, Wp2, bp2):
    N, D = x.shape
    E = edge_index.shape[1]

    row = edge_index[0]
    col = edge_index[1]
    row_r = row.reshape(NS, E // NS // CK, CK)
    col_r = col.reshape(NS, E // NS // CK, CK)
    row_flat = row.reshape(NW, E // NW)

    spmm = _spmm_fn(N, E)

    deg_parts = _deg_fn(N, E)(row_flat)
    dinv, g1 = _prep(deg_parts, x)

    p1 = spmm(g1, row_r, col_r)
    g2 = _layer1(p1, dinv, W1, b1)

    p2 = spmm(g2, row_r, col_r)
    return _final(p2, dinv, W2, b2, Wp1, bp1, Wp2, bp2)


# speedup vs baseline: 26.5414x; 1.0138x over previous
import functools

import jax
import jax.numpy as jnp
from jax import lax
from jax.experimental import pallas as pl
from jax.experimental.pallas import tpu as pltpu
from jax.experimental.pallas import tpu_sc as plsc

NC = 2
NS = 16
NW = NC * NS

CK = 80
HH = 64


def _mesh():
    return plsc.VectorSubcoreMesh(core_axis_name="c", subcore_axis_name="s")


@functools.cache
def _deg_fn(N, E):
    EPW = E // NW

    @functools.partial(
        pl.kernel, mesh=_mesh(),
        compiler_params=pltpu.CompilerParams(needs_layout_passes=False),
        out_type=jax.ShapeDtypeStruct((NW, N), jnp.float32),
        scratch_types=[
            pltpu.VMEM((EPW,), jnp.int32),
            pltpu.VMEM((N,), jnp.float32),
        ],
    )
    def deg_k(row_hbm, out_hbm, idx_v, deg_v):
        c = lax.axis_index("c")
        s = lax.axis_index("s")
        wid = s * NC + c
        pltpu.sync_copy(row_hbm.at[wid], idx_v)
        zero16 = jnp.zeros((16,), jnp.float32)

        def zbody(i, carry):
            deg_v[pl.ds(i * 16, 16)] = zero16
            return carry

        lax.fori_loop(0, N // 16, zbody, 0)
        ones16 = jnp.ones((16,), jnp.float32)

        def hbody(i, carry):
            idx16 = idx_v[pl.ds(i * 16, 16)]
            plsc.addupdate_scatter(deg_v, [idx16], ones16)
            return carry

        lax.fori_loop(0, EPW // 16, hbody, 0)
        pltpu.sync_copy(deg_v, out_hbm.at[wid])

    return deg_k


@functools.cache
def _spmm_fn(N, E):
    EPW = E // NS
    NCH = EPW // CK
    RPT = N // NS
    ZR = 125
    NZ = RPT // ZR

    @functools.partial(
        pl.kernel, mesh=_mesh(),
        compiler_params=pltpu.CompilerParams(
            needs_layout_passes=False, use_tc_tiling_on_sc=False),
        out_type=jax.ShapeDtypeStruct((NC, N, HH), jnp.float32),
        scratch_types=[
            pltpu.VMEM((NCH, CK), jnp.int32),
            pltpu.VMEM((NCH, CK), jnp.int32),
            [pltpu.VMEM((CK, HH), jnp.float32)] * 8,
            pltpu.VMEM((ZR, HH), jnp.float32),
            pltpu.VMEM_SHARED((N, HH), jnp.float32),
            [pltpu.SemaphoreType.DMA] * 8,
        ],
    )
    def spmm_k(g_hbm, row_hbm, col_hbm, out_hbm,
               row_v, col_v, rbufs, zbuf, acc, sems):
        c = lax.axis_index("c")
        s = lax.axis_index("s")
        pltpu.sync_copy(row_hbm.at[s], row_v)
        pltpu.sync_copy(col_hbm.at[s], col_v)
        ghalf = g_hbm.at[c]

        zero16 = jnp.zeros((16,), jnp.float32)

        def zrow(i, carry):
            def zcol(k, carry2):
                zbuf[i, pl.ds(k * 16, 16)] = zero16
                return carry2
            return lax.fori_loop(0, HH // 16, zcol, carry)

        lax.fori_loop(0, ZR, zrow, 0)

        def zacc(i, carry):
            pltpu.sync_copy(zbuf, acc.at[pl.ds(s * RPT + i * ZR, ZR)])
            return carry

        lax.fori_loop(0, NZ, zacc, 0)
        plsc.subcore_barrier()

        NB = 8

        def start(j, b):
            pltpu.async_copy(ghalf.at[col_v.at[j]], rbufs[b], sems[b])

        def finish(j, b):
            pltpu.make_async_copy(ghalf.at[col_v.at[j]], rbufs[b],
                                  sems[b]).wait()
            pltpu.sync_copy(rbufs[b], acc.at[row_v.at[j]], add=True)

        for b in range(NB):
            start(b, b)

        def ebody(i, carry):
            j0 = NB * i
            for b in range(NB):
                finish(j0 + b, b)

                @pl.when(j0 + b + NB < NCH)
                def _():
                    start(j0 + b + NB, b)

            return carry

        lax.fori_loop(0, NCH // NB, ebody, 0)
        for t in range(NCH - NCH % NB, NCH):
            finish(t, t % NB)
        plsc.subcore_barrier()
        DR = (N // NS) // 8 * 8
        pltpu.sync_copy(acc.at[pl.ds(s * DR, DR)],
                        out_hbm.at[c, pl.ds(s * DR, DR)])
        TAIL = N - NS * DR

        @pl.when(s == 0)
        def _():
            pltpu.sync_copy(acc.at[pl.ds(NS * DR, TAIL)],
                            out_hbm.at[c, pl.ds(NS * DR, TAIL)])

    return spmm_k


def _prep(parts, x):
    N, D = x.shape

    def body(parts_ref, x_ref, dinv_ref, g_ref):
        deg = jnp.sum(parts_ref[...], axis=0)
        dinv = lax.rsqrt(deg)
        dinv_ref[...] = dinv
        g = x_ref[...] * dinv[:, None]
        g_ref[0] = g[:, :HH]
        g_ref[1] = g[:, HH:]

    return pl.pallas_call(
        body,
        out_shape=[
            jax.ShapeDtypeStruct((N,), jnp.float32),
            jax.ShapeDtypeStruct((NC, N, HH), jnp.float32),
        ],
    )(parts, x)


def _layer1(parts, dinv, W1, b1):
    N = parts.shape[1]

    def body(p_ref, dinv_ref, W_ref, b_ref, g_ref):
        dv = dinv_ref[...][:, None]
        srows = jnp.concatenate([p_ref[0] * dv, p_ref[1] * dv], axis=1)
        h = jnp.dot(srows, W_ref[...], preferred_element_type=jnp.float32)
        h = jnp.maximum(h + b_ref[...][None, :], 0.0)
        g = h * dv
        g_ref[0] = g[:, :HH]
        g_ref[1] = g[:, HH:]

    return pl.pallas_call(
        body,
        out_shape=jax.ShapeDtypeStruct((NC, N, HH), jnp.float32),
    )(parts, dinv, W1, b1)


def _final(parts, dinv, W2, b2, Wp1, bp1, Wp2, bp2):
    N = parts.shape[1]
    O = Wp2.shape[1]

    def body(p_ref, dinv_ref, W2_ref, b2_ref,
             Wp1_ref, bp1_ref, Wp2_ref, bp2_ref, out_ref):
        dv = dinv_ref[...][:, None]
        srows = jnp.concatenate([p_ref[0] * dv, p_ref[1] * dv], axis=1)
        h = jnp.dot(srows, W2_ref[...], preferred_element_type=jnp.float32)
        h = jnp.maximum(h + b2_ref[...][None, :], 0.0)
        h = jnp.dot(h, Wp1_ref[...], preferred_element_type=jnp.float32)
        h = h + bp1_ref[...][None, :]
        h = jnp.dot(h, Wp2_ref[...], preferred_element_type=jnp.float32)
        h = h + bp2_ref[...][None, :]
        m = jnp.max(h, axis=1, keepdims=True)
        lse = jnp.log(jnp.sum(jnp.exp(h - m), axis=1, keepdims=True)) + m
        out_ref[...] = h - lse

    return pl.pallas_call(
        body,
        out_shape=jax.ShapeDtypeStruct((N, O), jnp.float32),
    )(parts, dinv, W2, b2, Wp1, bp1, Wp2, bp2)


def kernel(x, edge_index, W1, b1, W2, b2, Wp1, bp1, Wp2, bp2):
    N, D = x.shape
    E = edge_index.shape[1]

    row = edge_index[0]
    col = edge_index[1]
    row_r = row.reshape(NS, E // NS // CK, CK)
    col_r = col.reshape(NS, E // NS // CK, CK)
    row_flat = row.reshape(NW, E // NW)

    spmm = _spmm_fn(N, E)

    deg_parts = _deg_fn(N, E)(row_flat)
    dinv, g1 = _prep(deg_parts, x)

    p1 = spmm(g1, row_r, col_r)
    g2 = _layer1(p1, dinv, W1, b1)

    p2 = spmm(g2, row_r, col_r)
    return _final(p2, dinv, W2, b2, Wp1, bp1, Wp2, bp2)


# speedup vs baseline: 26.6866x; 1.0055x over previous
import functools

import jax
import jax.numpy as jnp
from jax import lax
from jax.experimental import pallas as pl
from jax.experimental.pallas import tpu as pltpu
from jax.experimental.pallas import tpu_sc as plsc

NC = 2
NS = 16
NW = NC * NS

CK = 80
HH = 64


def _mesh():
    return plsc.VectorSubcoreMesh(core_axis_name="c", subcore_axis_name="s")


@functools.cache
def _deg_fn(N, E):
    EPW = E // NW

    @functools.partial(
        pl.kernel, mesh=_mesh(),
        compiler_params=pltpu.CompilerParams(needs_layout_passes=False),
        out_type=jax.ShapeDtypeStruct((NW, N), jnp.float32),
        scratch_types=[
            pltpu.VMEM((EPW,), jnp.int32),
            pltpu.VMEM((N,), jnp.float32),
        ],
    )
    def deg_k(row_hbm, out_hbm, idx_v, deg_v):
        c = lax.axis_index("c")
        s = lax.axis_index("s")
        wid = s * NC + c
        pltpu.sync_copy(row_hbm.at[pl.ds(wid * EPW, EPW)], idx_v)
        zero16 = jnp.zeros((16,), jnp.float32)

        def zbody(i, carry):
            deg_v[pl.ds(i * 16, 16)] = zero16
            return carry

        lax.fori_loop(0, N // 16, zbody, 0)
        ones16 = jnp.ones((16,), jnp.float32)

        def hbody(i, carry):
            idx16 = idx_v[pl.ds(i * 16, 16)]
            plsc.addupdate_scatter(deg_v, [idx16], ones16)
            return carry

        lax.fori_loop(0, EPW // 16, hbody, 0)
        pltpu.sync_copy(deg_v, out_hbm.at[wid])

    return deg_k


@functools.cache
def _spmm_fn(N, E):
    EPW = E // NS
    NCH = EPW // CK
    RPT = N // NS
    ZR = 125
    NZ = RPT // ZR

    @functools.partial(
        pl.kernel, mesh=_mesh(),
        compiler_params=pltpu.CompilerParams(
            needs_layout_passes=False, use_tc_tiling_on_sc=False),
        out_type=jax.ShapeDtypeStruct((NC, N, HH), jnp.float32),
        scratch_types=[
            pltpu.VMEM((EPW,), jnp.int32),
            pltpu.VMEM((EPW,), jnp.int32),
            [pltpu.VMEM((CK, HH), jnp.float32)] * 8,
            pltpu.VMEM((ZR, HH), jnp.float32),
            pltpu.VMEM_SHARED((N, HH), jnp.float32),
            [pltpu.SemaphoreType.DMA] * 8,
        ],
    )
    def spmm_k(g_hbm, row_hbm, col_hbm, out_hbm,
               row_v, col_v, rbufs, zbuf, acc, sems):
        c = lax.axis_index("c")
        s = lax.axis_index("s")
        pltpu.sync_copy(row_hbm.at[pl.ds(s * EPW, EPW)], row_v)
        pltpu.sync_copy(col_hbm.at[pl.ds(s * EPW, EPW)], col_v)
        ghalf = g_hbm.at[c]

        zero16 = jnp.zeros((16,), jnp.float32)

        def zrow(i, carry):
            def zcol(k, carry2):
                zbuf[i, pl.ds(k * 16, 16)] = zero16
                return carry2
            return lax.fori_loop(0, HH // 16, zcol, carry)

        lax.fori_loop(0, ZR, zrow, 0)

        def zacc(i, carry):
            pltpu.sync_copy(zbuf, acc.at[pl.ds(s * RPT + i * ZR, ZR)])
            return carry

        lax.fori_loop(0, NZ, zacc, 0)
        plsc.subcore_barrier()

        NB = 8

        def start(j, b):
            pltpu.async_copy(ghalf.at[col_v.at[pl.ds(j * CK, CK)]], rbufs[b], sems[b])

        def finish(j, b):
            pltpu.make_async_copy(ghalf.at[col_v.at[pl.ds(j * CK, CK)]],
                                  rbufs[b], sems[b]).wait()
            pltpu.sync_copy(rbufs[b], acc.at[row_v.at[pl.ds(j * CK, CK)]],
                            add=True)

        for b in range(NB):
            start(b, b)

        def ebody(i, carry):
            j0 = NB * i
            for b in range(NB):
                finish(j0 + b, b)

                @pl.when(j0 + b + NB < NCH)
                def _():
                    start(j0 + b + NB, b)

            return carry

        lax.fori_loop(0, NCH // NB, ebody, 0)
        for t in range(NCH - NCH % NB, NCH):
            finish(t, t % NB)
        plsc.subcore_barrier()
        DR = (N // NS) // 8 * 8
        pltpu.sync_copy(acc.at[pl.ds(s * DR, DR)],
                        out_hbm.at[c, pl.ds(s * DR, DR)])
        TAIL = N - NS * DR

        @pl.when(s == 0)
        def _():
            pltpu.sync_copy(acc.at[pl.ds(NS * DR, TAIL)],
                            out_hbm.at[c, pl.ds(NS * DR, TAIL)])

    return spmm_k


def _prep(parts, x):
    N, D = x.shape

    def body(parts_ref, x_ref, dinv_ref, g_ref):
        deg = jnp.sum(parts_ref[...], axis=0)
        dinv = lax.rsqrt(deg)
        dinv_ref[...] = dinv
        g = x_ref[...] * dinv[:, None]
        g_ref[0] = g[:, :HH]
        g_ref[1] = g[:, HH:]

    return pl.pallas_call(
        body,
        out_shape=[
            jax.ShapeDtypeStruct((N,), jnp.float32),
            jax.ShapeDtypeStruct((NC, N, HH), jnp.float32),
        ],
    )(parts, x)


def _layer1(parts, dinv, W1, b1):
    N = parts.shape[1]

    def body(p_ref, dinv_ref, W_ref, b_ref, g_ref):
        dv = dinv_ref[...][:, None]
        srows = jnp.concatenate([p_ref[0] * dv, p_ref[1] * dv], axis=1)
        h = jnp.dot(srows, W_ref[...], preferred_element_type=jnp.float32)
        h = jnp.maximum(h + b_ref[...][None, :], 0.0)
        g = h * dv
        g_ref[0] = g[:, :HH]
        g_ref[1] = g[:, HH:]

    return pl.pallas_call(
        body,
        out_shape=jax.ShapeDtypeStruct((NC, N, HH), jnp.float32),
    )(parts, dinv, W1, b1)


def _final(parts, dinv, W2, b2, Wp1, bp1, Wp2, bp2):
    N = parts.shape[1]
    O = Wp2.shape[1]

    def body(p_ref, dinv_ref, W2_ref, b2_ref,
             Wp1_ref, bp1_ref, Wp2_ref, bp2_ref, out_ref):
        dv = dinv_ref[...][:, None]
        srows = jnp.concatenate([p_ref[0] * dv, p_ref[1] * dv], axis=1)
        h = jnp.dot(srows, W2_ref[...], preferred_element_type=jnp.float32)
        h = jnp.maximum(h + b2_ref[...][None, :], 0.0)
        h = jnp.dot(h, Wp1_ref[...], preferred_element_type=jnp.float32)
        h = h + bp1_ref[...][None, :]
        h = jnp.dot(h, Wp2_ref[...], preferred_element_type=jnp.float32)
        h = h + bp2_ref[...][None, :]
        m = jnp.max(h, axis=1, keepdims=True)
        lse = jnp.log(jnp.sum(jnp.exp(h - m), axis=1, keepdims=True)) + m
        out_ref[...] = h - lse

    return pl.pallas_call(
        body,
        out_shape=jax.ShapeDtypeStruct((N, O), jnp.float32),
    )(parts, dinv, W2, b2, Wp1, bp1, Wp2, bp2)


def kernel(x, edge_index, W1, b1, W2, b2, Wp1, bp1, Wp2, bp2):
    N, D = x.shape
    E = edge_index.shape[1]

    row = edge_index[0]
    col = edge_index[1]

    spmm = _spmm_fn(N, E)

    deg_parts = _deg_fn(N, E)(row)
    dinv, g1 = _prep(deg_parts, x)

    p1 = spmm(g1, row, col)
    g2 = _layer1(p1, dinv, W1, b1)

    p2 = spmm(g2, row, col)
    return _final(p2, dinv, W2, b2, Wp1, bp1, Wp2, bp2)
